# Initial kernel scaffold; baseline (speedup 1.0000x reference)
import jax
import jax.numpy as jnp
from jax.experimental import pallas as pl


def kernel(x, edge_index, edge_attr, prelu_w, W_enc, emb1, emb2, W1, b1, W2, b2):
    raise NotImplementedError("write your pallas kernel here")


# speedup vs baseline: 5.4302x; 5.4302x over previous
import functools

import jax
import jax.numpy as jnp
from jax import lax
from jax.experimental import pallas as pl
from jax.experimental.pallas import tpu as pltpu
from jax.experimental.pallas import tpu_sc as plsc

_NC = 2
_NS = 16
_NW = _NC * _NS
_L = 16
_K = 128
_IB = 8
_BN = 1000


def _enc_body(pw_ref, x_ref, wt_ref, h_ref):
    xb = x_ref[...]
    pw = pw_ref[0, 0]
    xa = jnp.where(xb > 0, xb, pw * xb)
    h_ref[...] = jnp.dot(xa, wt_ref[...], preferred_element_type=jnp.float32)


def _mlp_body(p0_ref, p1_ref, h_ref, c0_ref, c1_ref, embp_ref, w1t_ref,
              b1_ref, w2t_ref, b2_ref, out_ref):
    a = p0_ref[...] + p1_ref[...] + h_ref[...]
    cnt = c0_ref[...] + c1_ref[...]
    embp = embp_ref[...]
    a = a + jnp.dot(cnt, embp, preferred_element_type=jnp.float32)
    a = a + embp[12:13, :]
    hid = jnp.dot(a, w1t_ref[...], preferred_element_type=jnp.float32)
    hid = jnp.maximum(hid + b1_ref[...], 0.0)
    out_ref[...] = (jnp.dot(hid, w2t_ref[...],
                            preferred_element_type=jnp.float32) + b2_ref[...])


def _make_sc_kernel(n_chunks, nacc, cl, d):
    rows_pt = nacc // _NS
    cnt_pt = cl // _NS
    n_blocks = n_chunks // _IB
    mesh = plsc.VectorSubcoreMesh(core_axis_name="c", subcore_axis_name="s")

    @functools.partial(
        pl.kernel,
        mesh=mesh,
        out_type=[
            jax.ShapeDtypeStruct((_NC, nacc, d), jnp.float32),
            jax.ShapeDtypeStruct((_NC, cl), jnp.float32),
        ],
        scratch_types=[
            pltpu.VMEM((_IB, _K), jnp.int32),
            pltpu.VMEM((_IB, _K), jnp.int32),
            pltpu.VMEM((_IB, _K), jnp.int32),
            pltpu.VMEM((_K, d), jnp.float32),
            pltpu.VMEM((_K,), jnp.float32),
            pltpu.VMEM_SHARED((nacc, d), jnp.float32),
            pltpu.VMEM_SHARED((cl,), jnp.float32),
            pltpu.SemaphoreType.DMA,
        ],
    )
    def sc_kernel(h_hbm, srcp_hbm, dstp_hbm, cntp_hbm, zrow_hbm, zcnt_hbm,
                  aggr_out, cnt_out,
                  src_v, dst_v, cidx_v, rows_v, ones_v, aggr_sh, cnt_sh, sem):
        cid = lax.axis_index("c")
        sid = lax.axis_index("s")
        wid = sid * _NC + cid
        pltpu.sync_copy(zrow_hbm.at[pl.ds(sid * rows_pt, rows_pt)],
                        aggr_sh.at[pl.ds(sid * rows_pt, rows_pt)])
        pltpu.sync_copy(zcnt_hbm.at[pl.ds(sid * cnt_pt, cnt_pt)],
                        cnt_sh.at[pl.ds(sid * cnt_pt, cnt_pt)])
        for t in range(_K // _L):
            ones_v[pl.ds(t * _L, _L)] = jnp.ones((_L,), jnp.float32)
        plsc.subcore_barrier()

        def blk_step(b, carry):
            pltpu.sync_copy(srcp_hbm.at[wid].at[pl.ds(b * _IB, _IB)], src_v)
            pltpu.sync_copy(dstp_hbm.at[wid].at[pl.ds(b * _IB, _IB)], dst_v)
            pltpu.sync_copy(cntp_hbm.at[wid].at[pl.ds(b * _IB, _IB)], cidx_v)
            for j in range(_IB):
                pltpu.async_copy(h_hbm.at[src_v.at[j]], rows_v, sem).wait()
                pltpu.sync_copy(rows_v, aggr_sh.at[dst_v.at[j]], add=True)
                pltpu.sync_copy(ones_v, cnt_sh.at[cidx_v.at[j]], add=True)
            return carry

        lax.fori_loop(0, n_blocks, blk_step, 0)
        plsc.subcore_barrier()
        pltpu.sync_copy(aggr_sh.at[pl.ds(sid * rows_pt, rows_pt)],
                        aggr_out.at[cid].at[pl.ds(sid * rows_pt, rows_pt)])
        pltpu.sync_copy(cnt_sh.at[pl.ds(sid * cnt_pt, cnt_pt)],
                        cnt_out.at[cid].at[pl.ds(sid * cnt_pt, cnt_pt)])

    return sc_kernel


def kernel(x, edge_index, edge_attr, prelu_w, W_enc, emb1, emb2, W1, b1, W2, b2):
    n, d = x.shape
    e = edge_index.shape[1]
    f32, i32 = jnp.float32, jnp.int32

    n_chunks = -(-e // (_NW * _K * _IB)) * _IB
    e_pad = _NW * _K * n_chunks
    nacc = -(-(n + 1) // (_NS * 8)) * (_NS * 8)
    cl = -(-((n + 1) * 18) // (_NS * 128)) * (_NS * 128)

    grid = n // _BN
    h = pl.pallas_call(
        _enc_body,
        grid=(grid,),
        in_specs=[
            pl.BlockSpec((1, 1), lambda i: (0, 0)),
            pl.BlockSpec((_BN, d), lambda i: (i, 0)),
            pl.BlockSpec((d, d), lambda i: (0, 0)),
        ],
        out_specs=pl.BlockSpec((_BN, d), lambda i: (i, 0)),
        out_shape=jax.ShapeDtypeStruct((n, d), f32),
    )(prelu_w.reshape(1, 1), x, W_enc.T)

    src = edge_index[0]
    dst = edge_index[1]
    cidx = dst * 18 + edge_attr[:, 0] * 3 + edge_attr[:, 1]
    padn = e_pad - e
    srcp = jnp.concatenate([src, jnp.zeros((padn,), i32)]).reshape(_NW, n_chunks, _K)
    dstp = jnp.concatenate([dst, jnp.full((padn,), n, i32)]).reshape(_NW, n_chunks, _K)
    cntp = jnp.concatenate([cidx, jnp.full((padn,), n * 18, i32)]).reshape(_NW, n_chunks, _K)
    zrow = jnp.zeros((nacc, d), f32)
    zcnt = jnp.zeros((cl,), f32)

    aggr2, cnt2 = _make_sc_kernel(n_chunks, nacc, cl, d)(
        h, srcp, dstp, cntp, zrow, zcnt)

    p0 = lax.slice(aggr2[0], (0, 0), (n, d))
    p1 = lax.slice(aggr2[1], (0, 0), (n, d))
    c0 = lax.slice(cnt2[0], (0,), (n * 18,)).reshape(n, 18)
    c1 = lax.slice(cnt2[1], (0,), (n * 18,)).reshape(n, 18)

    emb18 = (emb1[:, None, :] + emb2[None, :, :]).reshape(18, d)

    out = pl.pallas_call(
        _mlp_body,
        grid=(grid,),
        in_specs=[
            pl.BlockSpec((_BN, d), lambda i: (i, 0)),
            pl.BlockSpec((_BN, d), lambda i: (i, 0)),
            pl.BlockSpec((_BN, d), lambda i: (i, 0)),
            pl.BlockSpec((_BN, 18), lambda i: (i, 0)),
            pl.BlockSpec((_BN, 18), lambda i: (i, 0)),
            pl.BlockSpec((18, d), lambda i: (0, 0)),
            pl.BlockSpec((d, 2 * d), lambda i: (0, 0)),
            pl.BlockSpec((1, 2 * d), lambda i: (0, 0)),
            pl.BlockSpec((2 * d, d), lambda i: (0, 0)),
            pl.BlockSpec((1, d), lambda i: (0, 0)),
        ],
        out_specs=pl.BlockSpec((_BN, d), lambda i: (i, 0)),
        out_shape=jax.ShapeDtypeStruct((n, d), f32),
    )(p0, p1, h, c0, c1, emb18, W1.T, b1.reshape(1, -1), W2.T, b2.reshape(1, -1))
    return out
